# baseline (speedup 1.0000x reference)
import functools

import jax
import jax.numpy as jnp
from jax import lax
from jax.experimental import pallas as pl
from jax.experimental.pallas import tpu as pltpu
from jax.experimental.pallas import tpu_sc as plsc

N_NODES = 100000
N_EDGES = 3200000
MAX_DEGREE = 512
EMBED_DIM = 128

NW = 32
N_PAD = 102400
NODES_PER_W = N_PAD // NW
EDGES_PER_W = N_EDGES // NW
ECHUNK = 4000
NECH = EDGES_PER_W // ECHUNK
NCHUNK = 128
NNCH = NODES_PER_W // NCHUNK

_mesh = plsc.VectorSubcoreMesh(core_axis_name="c", subcore_axis_name="s")
_params = pltpu.CompilerParams(needs_layout_passes=False)


@functools.partial(
    pl.kernel,
    out_type=[
        jax.ShapeDtypeStruct((NW, N_PAD), jnp.int32),
        jax.ShapeDtypeStruct((NW, N_PAD), jnp.int32),
    ],
    mesh=_mesh,
    compiler_params=_params,
    scratch_types=[
        pltpu.VMEM((N_PAD,), jnp.int32),
        pltpu.VMEM((ECHUNK,), jnp.int32),
    ],
)
def _hist_kernel(dst_hbm, src_hbm, in_part, out_part, hist, ebuf):
    wid = lax.axis_index("s") * 2 + lax.axis_index("c")
    ones = jnp.full((16,), 1, jnp.int32)
    ebase = pl.multiple_of(wid * EDGES_PER_W, 8)
    for which in range(2):
        ids_hbm = dst_hbm if which == 0 else src_hbm
        part = in_part if which == 0 else out_part

        @pl.loop(0, N_PAD // 16)
        def _zero(i):
            hist[pl.ds(i * 16, 16)] = jnp.zeros((16,), jnp.int32)

        @pl.loop(0, NECH)
        def _echunk(c):
            off = pl.multiple_of(ebase + c * ECHUNK, 8)
            pltpu.sync_copy(ids_hbm.at[pl.ds(off, ECHUNK)], ebuf)

            @pl.loop(0, ECHUNK // 16)
            def _scat(i):
                ids = ebuf[pl.ds(i * 16, 16)]
                plsc.addupdate_scatter(hist, [ids], ones)

        pltpu.sync_copy(hist, part.at[wid])


@functools.partial(
    pl.kernel,
    out_type=jax.ShapeDtypeStruct((N_PAD, EMBED_DIM), jnp.float32),
    mesh=_mesh,
    compiler_params=_params,
    scratch_types=[
        pltpu.VMEM((NODES_PER_W,), jnp.int32),
        pltpu.VMEM((NODES_PER_W,), jnp.int32),
        pltpu.VMEM((NW, NCHUNK), jnp.int32),
        pltpu.VMEM((NCHUNK, EMBED_DIM), jnp.float32),
        pltpu.VMEM((NCHUNK, EMBED_DIM), jnp.float32),
        pltpu.SemaphoreType.DMA,
        pltpu.SemaphoreType.DMA,
    ],
)
def _embed_kernel(in_part, out_part, w1_hbm, w2_hbm, out_hbm,
                  d1, d2, pbuf, rows_a, rows_b, sem_a, sem_b):
    wid = lax.axis_index("s") * 2 + lax.axis_index("c")
    nbase = pl.multiple_of(wid * NODES_PER_W, 8)

    for which in range(2):
        part = in_part if which == 0 else out_part
        dref = d1 if which == 0 else d2

        @pl.loop(0, NNCH)
        def _reduce(c):
            off = pl.multiple_of(nbase + c * NCHUNK, 8)
            pltpu.sync_copy(part.at[:, pl.ds(off, NCHUNK)], pbuf)
            for j in range(NCHUNK // 16):
                def _body(p, acc, j=j):
                    return acc + pbuf[p, pl.ds(j * 16, 16)]
                acc = lax.fori_loop(1, NW, _body, pbuf[0, pl.ds(j * 16, 16)])
                acc = jnp.minimum(acc, MAX_DEGREE)
                dref[pl.ds(c * NCHUNK + j * 16, 16)] = acc

    @pl.loop(0, NNCH)
    def _gather(c):
        coff = pl.multiple_of(c * NCHUNK, 8)
        idx1 = d1.at[pl.ds(coff, NCHUNK)]
        idx2 = d2.at[pl.ds(coff, NCHUNK)]
        cp_a = pltpu.async_copy(w1_hbm.at[idx1], rows_a, sem_a)
        cp_b = pltpu.async_copy(w2_hbm.at[idx2], rows_b, sem_b)
        cp_a.wait()
        cp_b.wait()

        @pl.loop(0, NCHUNK)
        def _add(r):
            for j in range(EMBED_DIM // 16):
                s = pl.ds(j * 16, 16)
                rows_a[r, s] = rows_a[r, s] + rows_b[r, s]

        pltpu.sync_copy(rows_a, out_hbm.at[pl.ds(nbase + coff, NCHUNK), :])


def kernel(edge_index, W1, W2):
    ei = edge_index.astype(jnp.int32)
    in_part, out_part = _hist_kernel(ei[1], ei[0])
    out = _embed_kernel(in_part, out_part, W1, W2)
    return out[:N_NODES]


# speedup vs baseline: 1.1360x; 1.1360x over previous
import functools

import jax
import jax.numpy as jnp
from jax import lax
from jax.experimental import pallas as pl
from jax.experimental.pallas import tpu as pltpu
from jax.experimental.pallas import tpu_sc as plsc

N_NODES = 100000
N_EDGES = 3200000
MAX_DEGREE = 512
EMBED_DIM = 128

NW = 32
N_PAD = 102400
NODES_PER_W = N_PAD // NW
EDGES_PER_W = N_EDGES // NW
ECHUNK = 4000
NECH = EDGES_PER_W // ECHUNK
NCHUNK = 128
NNCH = NODES_PER_W // NCHUNK
NVEC = 16

_mesh = plsc.VectorSubcoreMesh(core_axis_name="c", subcore_axis_name="s")
_params = pltpu.CompilerParams(needs_layout_passes=False)


@functools.partial(
    pl.kernel,
    out_type=[
        jax.ShapeDtypeStruct((NW, N_PAD), jnp.int32),
        jax.ShapeDtypeStruct((NW, N_PAD), jnp.int32),
    ],
    mesh=_mesh,
    compiler_params=_params,
    scratch_types=[
        pltpu.VMEM((N_PAD,), jnp.int32),
        pltpu.VMEM((ECHUNK,), jnp.int32),
        pltpu.VMEM((ECHUNK,), jnp.int32),
        pltpu.SemaphoreType.DMA,
        pltpu.SemaphoreType.DMA,
    ],
)
def _hist_kernel(dst_hbm, src_hbm, in_part, out_part,
                 hist, ebuf0, ebuf1, sem_e0, sem_e1):
    wid = lax.axis_index("s") * 2 + lax.axis_index("c")
    ones = jnp.full((NVEC,), 1, jnp.int32)
    zeros = jnp.zeros((NVEC,), jnp.int32)
    ebase = pl.multiple_of(wid * EDGES_PER_W, 8)
    ebufs = [ebuf0, ebuf1]
    sems = [sem_e0, sem_e1]
    for which in range(2):
        ids_hbm = dst_hbm if which == 0 else src_hbm
        part = in_part if which == 0 else out_part

        def _issue(c):
            p = c & 1
            off = pl.multiple_of(ebase + c * ECHUNK, 8)
            return pltpu.async_copy(
                ids_hbm.at[pl.ds(off, ECHUNK)], ebufs[p], sems[p])

        descs = [None, None]
        descs[0] = _issue(0)
        descs[1] = _issue(1)

        @pl.loop(0, N_PAD // NVEC, unroll=8)
        def _zero(i):
            hist[pl.ds(i * NVEC, NVEC)] = zeros

        for c in range(NECH):
            p = c & 1
            descs[p].wait()
            ebuf = ebufs[p]

            @pl.loop(0, ECHUNK // NVEC, unroll=8)
            def _scat(i, ebuf=ebuf):
                ids = ebuf[pl.ds(i * NVEC, NVEC)]
                plsc.addupdate_scatter(hist, [ids], ones)

            if c + 2 < NECH:
                descs[p] = _issue(c + 2)

        pltpu.sync_copy(hist, part.at[wid])


@functools.partial(
    pl.kernel,
    out_type=jax.ShapeDtypeStruct((N_PAD, EMBED_DIM), jnp.float32),
    mesh=_mesh,
    compiler_params=_params,
    scratch_types=[
        pltpu.VMEM((NODES_PER_W,), jnp.int32),
        pltpu.VMEM((NODES_PER_W,), jnp.int32),
        pltpu.VMEM((NW, NCHUNK), jnp.int32),
        [pltpu.VMEM((NCHUNK, EMBED_DIM), jnp.float32) for _ in range(3)],
        [pltpu.VMEM((NCHUNK, EMBED_DIM), jnp.float32) for _ in range(3)],
        [pltpu.SemaphoreType.DMA for _ in range(3)],
        [pltpu.SemaphoreType.DMA for _ in range(3)],
        [pltpu.SemaphoreType.DMA for _ in range(3)],
    ],
)
def _embed_kernel(in_part, out_part, w1_hbm, w2_hbm, out_hbm,
                  d1, d2, pbuf, rows_a, rows_b, sem_a, sem_b, sem_w):
    wid = lax.axis_index("s") * 2 + lax.axis_index("c")
    nbase = pl.multiple_of(wid * NODES_PER_W, 8)

    for which in range(2):
        part = in_part if which == 0 else out_part
        dref = d1 if which == 0 else d2

        @pl.loop(0, NNCH)
        def _reduce(c, part=part, dref=dref):
            off = pl.multiple_of(nbase + c * NCHUNK, 8)
            pltpu.sync_copy(part.at[:, pl.ds(off, NCHUNK)], pbuf)

            @pl.loop(0, NCHUNK // NVEC)
            def _vec(j, dref=dref, c=c):
                s = pl.ds(j * NVEC, NVEC)
                acc = pbuf[0, s]
                for p in range(1, NW):
                    acc = acc + pbuf[p, s]
                acc = jnp.minimum(acc, MAX_DEGREE)
                dref[pl.ds(c * NCHUNK + j * NVEC, NVEC)] = acc

    ga = [None, None, None]
    gb = [None, None, None]
    wd = [None, None, None]

    def _issue(c):
        p = c % 3
        if wd[p] is not None:
            wd[p].wait()
            wd[p] = None
        idx1 = d1.at[pl.ds(c * NCHUNK, NCHUNK)]
        idx2 = d2.at[pl.ds(c * NCHUNK, NCHUNK)]
        ga[p] = pltpu.async_copy(w1_hbm.at[idx1], rows_a[p], sem_a[p])
        gb[p] = pltpu.async_copy(w2_hbm.at[idx2], rows_b[p], sem_b[p])

    _issue(0)
    _issue(1)
    for c in range(NNCH):
        p = c % 3
        ga[p].wait()
        gb[p].wait()
        ra, rb = rows_a[p], rows_b[p]

        @pl.loop(0, NCHUNK)
        def _add(r, ra=ra, rb=rb):
            for j in range(EMBED_DIM // NVEC):
                s = pl.ds(j * NVEC, NVEC)
                plsc.addupdate(ra.at[r, s], rb[r, s])

        wd[p] = pltpu.async_copy(
            ra, out_hbm.at[pl.ds(nbase + c * NCHUNK, NCHUNK), :], sem_w[p])
        if c + 2 < NNCH:
            _issue(c + 2)
    for p in range(3):
        if wd[p] is not None:
            wd[p].wait()


def kernel(edge_index, W1, W2):
    ei = edge_index.astype(jnp.int32)
    in_part, out_part = _hist_kernel(ei[1], ei[0])
    out = _embed_kernel(in_part, out_part, W1, W2)
    return out[:N_NODES]


# speedup vs baseline: 2.6784x; 2.3578x over previous
import functools

import jax
import jax.numpy as jnp
from jax import lax
from jax.experimental import pallas as pl
from jax.experimental.pallas import tpu as pltpu
from jax.experimental.pallas import tpu_sc as plsc

N_NODES = 100000
N_EDGES = 3200000
MAX_DEGREE = 512
EMBED_DIM = 128
N_ROWS = MAX_DEGREE + 1

NW = 32
N_PAD = 102400
NODES_PER_W = N_PAD // NW
EDGES_PER_W = N_EDGES // NW
ECHUNK = 4000
NECH = EDGES_PER_W // ECHUNK
NCHUNK = 128
NNCH = NODES_PER_W // NCHUNK
RCHUNK = 400
NRCH = NODES_PER_W // RCHUNK
NVEC = 16

_mesh = plsc.VectorSubcoreMesh(core_axis_name="c", subcore_axis_name="s")
_params = pltpu.CompilerParams(
    needs_layout_passes=False, use_tc_tiling_on_sc=False)


@functools.partial(
    pl.kernel,
    out_type=[
        jax.ShapeDtypeStruct((NW, NW, NODES_PER_W), jnp.int32),
        jax.ShapeDtypeStruct((NW, NW, NODES_PER_W), jnp.int32),
    ],
    mesh=_mesh,
    compiler_params=_params,
    scratch_types=[
        pltpu.VMEM((N_PAD,), jnp.int32),
        pltpu.VMEM((ECHUNK,), jnp.int32),
        pltpu.VMEM((ECHUNK,), jnp.int32),
        pltpu.SemaphoreType.DMA,
        pltpu.SemaphoreType.DMA,
        pltpu.SemaphoreType.DMA,
    ],
)
def _hist_kernel(dst_hbm, src_hbm, in_part, out_part,
                 hist, ebuf0, ebuf1, sem_e0, sem_e1, sem_w):
    wid = lax.axis_index("s") * 2 + lax.axis_index("c")
    ones = jnp.full((NVEC,), 1, jnp.int32)
    zeros = jnp.zeros((NVEC,), jnp.int32)
    ebase = pl.multiple_of(wid * EDGES_PER_W, 8)
    ebufs = [ebuf0, ebuf1]
    sems = [sem_e0, sem_e1]
    for which in range(2):
        ids_hbm = dst_hbm if which == 0 else src_hbm
        part = in_part if which == 0 else out_part

        def _issue(c):
            p = c & 1
            off = pl.multiple_of(ebase + c * ECHUNK, 8)
            return pltpu.async_copy(
                ids_hbm.at[pl.ds(off, ECHUNK)], ebufs[p], sems[p])

        descs = [None, None]
        descs[0] = _issue(0)
        descs[1] = _issue(1)

        @pl.loop(0, N_PAD // NVEC, unroll=8)
        def _zero(i):
            hist[pl.ds(i * NVEC, NVEC)] = zeros

        for c in range(NECH):
            p = c & 1
            descs[p].wait()
            ebuf = ebufs[p]

            @pl.loop(0, ECHUNK // NVEC, unroll=8)
            def _scat(i, ebuf=ebuf):
                ids = ebuf[pl.ds(i * NVEC, NVEC)]
                plsc.addupdate_scatter(hist, [ids], ones)

            if c + 2 < NECH:
                descs[p] = _issue(c + 2)

        wds = []
        for o in range(NW):
            wds.append(pltpu.async_copy(
                hist.at[pl.ds(o * NODES_PER_W, NODES_PER_W)],
                part.at[o, wid], sem_w))
        for wd in wds:
            wd.wait()


@functools.partial(
    pl.kernel,
    out_type=jax.ShapeDtypeStruct((N_PAD, EMBED_DIM), jnp.float32),
    mesh=_mesh,
    compiler_params=_params,
    scratch_types=[
        pltpu.VMEM((N_ROWS * EMBED_DIM,), jnp.bfloat16),
        pltpu.VMEM((N_ROWS * EMBED_DIM,), jnp.bfloat16),
        pltpu.VMEM((NODES_PER_W,), jnp.int32),
        pltpu.VMEM((NODES_PER_W,), jnp.int32),
        pltpu.VMEM((NW, RCHUNK), jnp.int32),
        [pltpu.VMEM((NCHUNK, EMBED_DIM), jnp.float32) for _ in range(2)],
        pltpu.SemaphoreType.DMA,
        [pltpu.SemaphoreType.DMA for _ in range(2)],
    ],
)
def _embed_kernel(in_part, out_part, w1_hbm, w2_hbm, out_hbm,
                  w1b, w2b, d1, d2, pstage, rows, sem_t, sem_w):
    wid = lax.axis_index("s") * 2 + lax.axis_index("c")
    nbase = pl.multiple_of(wid * NODES_PER_W, 8)

    td1 = pltpu.async_copy(w1_hbm, w1b, sem_t)
    td2 = pltpu.async_copy(w2_hbm, w2b, sem_t)

    for which in range(2):
        part = in_part if which == 0 else out_part
        dref = d1 if which == 0 else d2
        for k in range(NRCH):
            pltpu.sync_copy(part.at[wid, :, pl.ds(k * RCHUNK, RCHUNK)],
                            pstage)

            @pl.loop(0, RCHUNK // NVEC)
            def _vec(j, dref=dref, k=k):
                s = pl.ds(j * NVEC, NVEC)
                acc = pstage[0, s]
                for p in range(1, NW):
                    acc = acc + pstage[p, s]
                acc = jnp.minimum(acc, MAX_DEGREE)
                dref[pl.ds(k * RCHUNK + j * NVEC, NVEC)] = acc

    td1.wait()
    td2.wait()

    def _drain(p):
        pltpu.make_async_copy(
            rows[p], out_hbm.at[pl.ds(nbase, NCHUNK), :], sem_w[p]).wait()

    def _do_chunk(chunk, p, guard):
        cbase = pl.multiple_of(chunk * NCHUNK, 8)
        rbuf = rows[p]
        if guard is None:
            _drain(p)
        else:
            @pl.when(guard)
            def _():
                _drain(p)

        @pl.loop(0, NCHUNK // NVEC)
        def _grp(g, rbuf=rbuf, cbase=cbase):
            dv1 = d1[pl.ds(cbase + g * NVEC, NVEC)]
            dv2 = d2[pl.ds(cbase + g * NVEC, NVEC)]
            for t in range(NVEC):
                off1 = dv1[t] * EMBED_DIM
                off2 = dv2[t] * EMBED_DIM
                r = g * NVEC + t
                for j in range(EMBED_DIM // 32):
                    wa = w1b[pl.ds(off1 + j * 32, 32)]
                    wb = w2b[pl.ds(off2 + j * 32, 32)]
                    a_lo, a_hi = plsc.unpack(
                        wa, format=plsc.PackFormat.INTERLEAVED)
                    b_lo, b_hi = plsc.unpack(
                        wb, format=plsc.PackFormat.INTERLEAVED)
                    rbuf[r, pl.ds(j * 32, NVEC)] = a_lo + b_lo
                    rbuf[r, pl.ds(j * 32 + NVEC, NVEC)] = a_hi + b_hi

        pltpu.async_copy(
            rbuf, out_hbm.at[pl.ds(nbase + cbase, NCHUNK), :], sem_w[p])

    @pl.loop(0, NNCH // 2)
    def _pair(q):
        _do_chunk(2 * q, 0, q > 0)
        _do_chunk(2 * q + 1, 1, q > 0)

    _do_chunk(NNCH - 1, 0, None)
    _drain(0)
    _drain(1)


def _interleave(w):
    t = w.astype(jnp.bfloat16).reshape(N_ROWS, EMBED_DIM // 32, 2, NVEC)
    return t.transpose(0, 1, 3, 2).reshape(-1)


def kernel(edge_index, W1, W2):
    ei = edge_index.astype(jnp.int32)
    in_part, out_part = _hist_kernel(ei[1], ei[0])
    out = _embed_kernel(in_part, out_part, _interleave(W1), _interleave(W2))
    return out[:N_NODES]


# speedup vs baseline: 3.1291x; 1.1683x over previous
import functools

import jax
import jax.numpy as jnp
from jax import lax
from jax.experimental import pallas as pl
from jax.experimental.pallas import tpu as pltpu
from jax.experimental.pallas import tpu_sc as plsc

N_NODES = 100000
N_EDGES = 3200000
MAX_DEGREE = 512
EMBED_DIM = 128
N_ROWS = MAX_DEGREE + 1

NW = 32
N_PAD = 102400
NODES_PER_W = N_PAD // NW
EDGES_PER_W = N_EDGES // NW
ECHUNK = 4000
NECH = EDGES_PER_W // ECHUNK
NCHUNK = 128
NNCH = NODES_PER_W // NCHUNK
RCHUNK = 400
NRCH = NODES_PER_W // RCHUNK
NVEC = 16
NTAIL = N_NODES % NCHUNK

_mesh = plsc.VectorSubcoreMesh(core_axis_name="c", subcore_axis_name="s")
_params = pltpu.CompilerParams(
    needs_layout_passes=False, use_tc_tiling_on_sc=False)


@functools.partial(
    pl.kernel,
    out_type=[
        jax.ShapeDtypeStruct((NW, NW, NODES_PER_W), jnp.int32),
        jax.ShapeDtypeStruct((NW, NW, NODES_PER_W), jnp.int32),
    ],
    mesh=_mesh,
    compiler_params=_params,
    scratch_types=[
        pltpu.VMEM((N_PAD,), jnp.int32),
        pltpu.VMEM((ECHUNK,), jnp.int32),
        pltpu.VMEM((ECHUNK,), jnp.int32),
        pltpu.SemaphoreType.DMA,
        pltpu.SemaphoreType.DMA,
        pltpu.SemaphoreType.DMA,
    ],
)
def _hist_kernel(edge_hbm, in_part, out_part,
                 hist, ebuf0, ebuf1, sem_e0, sem_e1, sem_w):
    wid = lax.axis_index("s") * 2 + lax.axis_index("c")
    ones = jnp.full((NVEC,), 1, jnp.int32)
    zeros = jnp.zeros((NVEC,), jnp.int32)
    ebase = pl.multiple_of(wid * EDGES_PER_W, 8)
    ebufs = [ebuf0, ebuf1]
    sems = [sem_e0, sem_e1]
    for which in range(2):
        row = 1 - which
        part = in_part if which == 0 else out_part

        def _issue(c, row=row):
            p = c & 1
            off = pl.multiple_of(ebase + c * ECHUNK, 8)
            return pltpu.async_copy(
                edge_hbm.at[row, pl.ds(off, ECHUNK)], ebufs[p], sems[p])

        descs = [None, None]
        descs[0] = _issue(0)
        descs[1] = _issue(1)

        @pl.loop(0, N_PAD // NVEC, unroll=8)
        def _zero(i):
            hist[pl.ds(i * NVEC, NVEC)] = zeros

        for c in range(NECH):
            p = c & 1
            descs[p].wait()
            ebuf = ebufs[p]

            @pl.loop(0, ECHUNK // NVEC, unroll=8)
            def _scat(i, ebuf=ebuf):
                ids = ebuf[pl.ds(i * NVEC, NVEC)]
                plsc.addupdate_scatter(hist, [ids], ones)

            if c + 2 < NECH:
                descs[p] = _issue(c + 2)

        wds = []
        for o in range(NW):
            wds.append(pltpu.async_copy(
                hist.at[pl.ds(o * NODES_PER_W, NODES_PER_W)],
                part.at[o, wid], sem_w))
        for wd in wds:
            wd.wait()


@functools.partial(
    pl.kernel,
    out_type=jax.ShapeDtypeStruct((N_NODES, EMBED_DIM), jnp.float32),
    mesh=_mesh,
    compiler_params=_params,
    scratch_types=[
        pltpu.VMEM((N_ROWS * EMBED_DIM,), jnp.bfloat16),
        pltpu.VMEM((N_ROWS * EMBED_DIM,), jnp.bfloat16),
        pltpu.VMEM((NODES_PER_W,), jnp.int32),
        pltpu.VMEM((NODES_PER_W,), jnp.int32),
        [pltpu.VMEM((NW, RCHUNK), jnp.int32) for _ in range(2)],
        [pltpu.VMEM((NCHUNK, EMBED_DIM), jnp.float32) for _ in range(2)],
        pltpu.SemaphoreType.DMA,
        [pltpu.SemaphoreType.DMA for _ in range(2)],
        [pltpu.SemaphoreType.DMA for _ in range(2)],
    ],
)
def _embed_kernel(in_part, out_part, w1_hbm, w2_hbm, out_hbm,
                  w1b, w2b, d1, d2, pstage, rows, sem_t, sem_p, sem_w):
    wid = lax.axis_index("s") * 2 + lax.axis_index("c")
    nbase = pl.multiple_of(wid * NODES_PER_W, 8)

    td1 = pltpu.async_copy(w1_hbm, w1b, sem_t)
    td2 = pltpu.async_copy(w2_hbm, w2b, sem_t)

    def _pissue(which, k):
        part = in_part if which == 0 else out_part
        p = k & 1
        return pltpu.async_copy(
            part.at[wid, :, pl.ds(k * RCHUNK, RCHUNK)], pstage[p], sem_p[p])

    pd = [None, None]
    pd[0] = _pissue(0, 0)
    pd[1] = _pissue(0, 1)
    for which in range(2):
        dref = d1 if which == 0 else d2
        for k in range(NRCH):
            p = k & 1
            pd[p].wait()
            buf = pstage[p]

            @pl.loop(0, RCHUNK // NVEC)
            def _vec(j, dref=dref, k=k, buf=buf):
                s = pl.ds(j * NVEC, NVEC)
                acc = buf[0, s]
                for q in range(1, NW):
                    acc = acc + buf[q, s]
                acc = jnp.minimum(acc, MAX_DEGREE)
                dref[pl.ds(k * RCHUNK + j * NVEC, NVEC)] = acc

            nxt = k + 2
            if nxt < NRCH:
                pd[p] = _pissue(which, nxt)
            elif which == 0:
                pd[p] = _pissue(1, nxt - NRCH)

    td1.wait()
    td2.wait()

    def _drain(p, chunk):
        left = N_NODES - nbase - chunk * NCHUNK

        @pl.when(left >= NCHUNK)
        def _():
            pltpu.make_async_copy(
                rows[p], out_hbm.at[pl.ds(0, NCHUNK), :], sem_w[p]).wait()

        @pl.when((left > 0) & (left < NCHUNK))
        def _():
            pltpu.make_async_copy(
                rows[p].at[pl.ds(0, NTAIL), :],
                out_hbm.at[pl.ds(0, NTAIL), :], sem_w[p]).wait()

    def _do_chunk(chunk, p, guard):
        cbase = pl.multiple_of(chunk * NCHUNK, 8)
        rbuf = rows[p]
        if guard is None:
            _drain(p, chunk - 2)
        else:
            @pl.when(guard)
            def _():
                _drain(p, chunk - 2)

        @pl.loop(0, NCHUNK // NVEC)
        def _grp(g, rbuf=rbuf, cbase=cbase):
            dv1 = d1[pl.ds(cbase + g * NVEC, NVEC)]
            dv2 = d2[pl.ds(cbase + g * NVEC, NVEC)]
            for t in range(NVEC):
                off1 = dv1[t] * EMBED_DIM
                off2 = dv2[t] * EMBED_DIM
                r = g * NVEC + t
                for j in range(EMBED_DIM // 32):
                    wa = w1b[pl.ds(off1 + j * 32, 32)]
                    wb = w2b[pl.ds(off2 + j * 32, 32)]
                    a_lo, a_hi = plsc.unpack(
                        wa, format=plsc.PackFormat.INTERLEAVED)
                    b_lo, b_hi = plsc.unpack(
                        wb, format=plsc.PackFormat.INTERLEAVED)
                    rbuf[r, pl.ds(j * 32, NVEC)] = a_lo + b_lo
                    rbuf[r, pl.ds(j * 32 + NVEC, NVEC)] = a_hi + b_hi

        start = nbase + cbase
        left = N_NODES - start

        @pl.when(left >= NCHUNK)
        def _():
            pltpu.async_copy(
                rbuf, out_hbm.at[pl.ds(start, NCHUNK), :], sem_w[p])

        @pl.when((left > 0) & (left < NCHUNK))
        def _():
            pltpu.async_copy(
                rbuf.at[pl.ds(0, NTAIL), :],
                out_hbm.at[pl.ds(start, NTAIL), :], sem_w[p])

    @pl.loop(0, NNCH // 2)
    def _pair(q):
        _do_chunk(2 * q, 0, q > 0)
        _do_chunk(2 * q + 1, 1, q > 0)

    _do_chunk(NNCH - 1, 0, None)
    _drain(0, NNCH - 1)
    _drain(1, NNCH - 2)


def _interleave(w):
    t = w.astype(jnp.bfloat16).reshape(N_ROWS, EMBED_DIM // 32, 2, NVEC)
    return t.transpose(0, 1, 3, 2).reshape(-1)


def kernel(edge_index, W1, W2):
    ei = edge_index.astype(jnp.int32)
    in_part, out_part = _hist_kernel(ei)
    return _embed_kernel(in_part, out_part, _interleave(W1), _interleave(W2))
